# baseline (speedup 1.0000x reference)
import functools

import jax
import jax.numpy as jnp
from jax import lax
from jax.experimental import pallas as pl
from jax.experimental.pallas import tpu as pltpu
from jax.experimental.pallas import tpu_sc as plsc

N_NODES = 10000
N_EDGES = 320000
D = 128
H = 64

T_PAD = 10240
A_PAD = 10048
PAD_IDX = N_NODES
CHUNK = 128
N_CHUNKS = 157
PER_TILE = N_CHUNKS * CHUNK
E_PAD = 16 * PER_TILE
ROWS_PER_TILE = A_PAD // 16
SLAB = ROWS_PER_TILE // 4
DEG_RUN = A_PAD // 4


def _tc_project(x_pad, w, b):
    blk = 1024

    def body(x_ref, w_ref, b_ref, o_ref):
        whaf = jnp.dot(x_ref[...], w_ref[0],
                       preferred_element_type=jnp.float32) + b_ref[0:1, :]
        whab = jnp.dot(x_ref[...], w_ref[1],
                       preferred_element_type=jnp.float32) + b_ref[1:2, :]
        o_ref[0] = whaf[:, 0:H]
        o_ref[1] = whaf[:, H:D]
        o_ref[2] = whab[:, 0:H]
        o_ref[3] = whab[:, H:D]

    return pl.pallas_call(
        body,
        grid=(T_PAD // blk,),
        in_specs=[
            pl.BlockSpec((blk, D), lambda i: (i, 0)),
            pl.BlockSpec((2, D, D), lambda i: (0, 0, 0)),
            pl.BlockSpec((2, D), lambda i: (0, 0)),
        ],
        out_specs=pl.BlockSpec((4, blk, H), lambda i: (0, i, 0)),
        out_shape=jax.ShapeDtypeStruct((4, T_PAD, H), jnp.float32),
    )(x_pad, w, b)


def _sc_aggregate(wh_flat, ef, eb, src_off, zrow, zcnt, ones):
    mesh = plsc.VectorSubcoreMesh(core_axis_name="c", subcore_axis_name="s")

    @functools.partial(
        pl.kernel,
        out_type=[
            jax.ShapeDtypeStruct((2, A_PAD, H), jnp.float32),
            jax.ShapeDtypeStruct((2, A_PAD), jnp.float32),
        ],
        mesh=mesh,
        compiler_params=pltpu.CompilerParams(use_tc_tiling_on_sc=False),
        scratch_types=[
            pltpu.VMEM((N_CHUNKS, CHUNK), jnp.int32),
            pltpu.VMEM((N_CHUNKS, CHUNK), jnp.int32),
            pltpu.VMEM((N_CHUNKS, CHUNK), jnp.int32),
            pltpu.VMEM((CHUNK, H), jnp.float32),
            pltpu.VMEM((SLAB, H), jnp.float32),
            pltpu.VMEM((DEG_RUN,), jnp.float32),
            pltpu.VMEM((CHUNK,), jnp.float32),
            pltpu.VMEM_SHARED((A_PAD, H), jnp.float32),
            pltpu.VMEM_SHARED((A_PAD,), jnp.float32),
            pltpu.SemaphoreType.DMA,
        ],
    )
    def agg(wh_hbm, ef_hbm, eb_hbm, srcoff_hbm, zrow_hbm, zcnt_hbm,
            ones_hbm, g_out, deg_out,
            srcg_v, dst_v, srcr_v, rows_v, zbuf, cbuf, ones_v,
            acc_sh, deg_sh, sem):
        c = lax.axis_index("c")
        s = lax.axis_index("s")

        pltpu.sync_copy(zrow_hbm, zbuf)
        pltpu.sync_copy(zcnt_hbm, cbuf)
        pltpu.sync_copy(ones_hbm, ones_v)

        base = s * ROWS_PER_TILE
        for k in range(4):
            off = base + k * SLAB
            pltpu.sync_copy(zbuf, acc_sh.at[pl.ds(off, SLAB)])

        @pl.when(s == 0)
        def _():
            for k in range(4):
                pltpu.sync_copy(cbuf, deg_sh.at[pl.ds(k * DEG_RUN, DEG_RUN)])
        plsc.subcore_barrier()

        for t in range(2):
            e_hbm = ef_hbm if t == 0 else eb_hbm
            pltpu.sync_copy(srcoff_hbm.at[c, t, s], srcg_v)
            pltpu.sync_copy(e_hbm.at[1, s], dst_v)

            def chunk_body(j, carry):
                pltpu.async_copy(wh_hbm.at[srcg_v.at[j]], rows_v, sem).wait()
                pltpu.sync_copy(rows_v, acc_sh.at[dst_v.at[j]], add=True)
                return carry

            lax.fori_loop(0, N_CHUNKS, chunk_body, 0)

            @pl.when(c == t)
            def _():
                pltpu.sync_copy(e_hbm.at[0, s], srcr_v)

                def deg_body(j, carry):
                    pltpu.sync_copy(ones_v, deg_sh.at[srcr_v.at[j]],
                                    add=True)
                    return carry

                lax.fori_loop(0, N_CHUNKS, deg_body, 0)

        plsc.subcore_barrier()

        for k in range(4):
            off = base + k * SLAB
            pltpu.sync_copy(acc_sh.at[pl.ds(off, SLAB)],
                            g_out.at[c, pl.ds(off, SLAB)])

        @pl.when(s % 4 == 0)
        def _():
            run = pl.multiple_of((s // 4) * DEG_RUN, DEG_RUN)
            pltpu.sync_copy(deg_sh.at[pl.ds(run, DEG_RUN)],
                            deg_out.at[c, pl.ds(run, DEG_RUN)])

    return agg(wh_flat, ef, eb, src_off, zrow, zcnt, ones)


def _tc_finalize(g, deg):
    blk = 1000

    def body(g_ref, deg_ref, o_ref):
        d = deg_ref[0] + deg_ref[1]
        inv = 1.0 / jnp.where(d == 0.0, 1.0, d)
        acc = jnp.concatenate([g_ref[0], g_ref[1]], axis=1) * inv
        o_ref[...] = jnp.maximum(acc, 0.0)

    return pl.pallas_call(
        body,
        grid=(N_NODES // blk,),
        in_specs=[
            pl.BlockSpec((2, blk, H), lambda i: (0, i, 0)),
            pl.BlockSpec((2, blk, 1), lambda i: (0, i, 0)),
        ],
        out_specs=pl.BlockSpec((blk, D), lambda i: (i, 0)),
        out_shape=jax.ShapeDtypeStruct((N_NODES, D), jnp.float32),
    )(g, deg)


def _prep_edges(ei):
    p = jnp.pad(ei, ((0, 0), (0, E_PAD - N_EDGES)), constant_values=PAD_IDX)
    return p.reshape(2, 16, N_CHUNKS, CHUNK)


@jax.jit
def kernel(x, edge_index_fwd, edge_index_bwd, W_af, b_af, W_ab, b_ab):
    x_pad = jnp.pad(x, ((0, T_PAD - N_NODES), (0, 0)))
    ef = _prep_edges(edge_index_fwd)
    eb = _prep_edges(edge_index_bwd)
    src_off = jnp.stack([
        jnp.stack([ef[0] + 0 * T_PAD, eb[0] + 2 * T_PAD]),
        jnp.stack([ef[0] + 1 * T_PAD, eb[0] + 3 * T_PAD]),
    ])
    zrow = jnp.zeros((SLAB, H), jnp.float32)
    zcnt = jnp.zeros((DEG_RUN,), jnp.float32)
    ones = jnp.ones((CHUNK,), jnp.float32)

    w = jnp.stack([W_af, W_ab])
    b = jnp.stack([b_af, b_ab])
    wh = _tc_project(x_pad, w, b).reshape(4 * T_PAD, H)

    g, deg = _sc_aggregate(wh, ef, eb, src_off, zrow, zcnt, ones)
    return _tc_finalize(g, deg.reshape(2, A_PAD, 1))


# speedup vs baseline: 1.0140x; 1.0140x over previous
import functools

import jax
import jax.numpy as jnp
from jax import lax
from jax.experimental import pallas as pl
from jax.experimental.pallas import tpu as pltpu
from jax.experimental.pallas import tpu_sc as plsc

N_NODES = 10000
N_EDGES = 320000
D = 128
H = 64

T_PAD = 10240
A_PAD = 10048
PAD_IDX = N_NODES
CHUNK = 128
N_CHUNKS = 157
PER_TILE = N_CHUNKS * CHUNK
E_PAD = 16 * PER_TILE
ROWS_PER_TILE = A_PAD // 16
SLAB = ROWS_PER_TILE // 4
DEG_RUN = A_PAD // 4


def _tc_project(x_pad, w, b):
    blk = 1024

    def body(x_ref, w_ref, b_ref, o_ref):
        whaf = jnp.dot(x_ref[...], w_ref[0],
                       preferred_element_type=jnp.float32) + b_ref[0:1, :]
        whab = jnp.dot(x_ref[...], w_ref[1],
                       preferred_element_type=jnp.float32) + b_ref[1:2, :]
        o_ref[0] = whaf[:, 0:H]
        o_ref[1] = whaf[:, H:D]
        o_ref[2] = whab[:, 0:H]
        o_ref[3] = whab[:, H:D]

    return pl.pallas_call(
        body,
        grid=(T_PAD // blk,),
        in_specs=[
            pl.BlockSpec((blk, D), lambda i: (i, 0)),
            pl.BlockSpec((2, D, D), lambda i: (0, 0, 0)),
            pl.BlockSpec((2, D), lambda i: (0, 0)),
        ],
        out_specs=pl.BlockSpec((4, blk, H), lambda i: (0, i, 0)),
        out_shape=jax.ShapeDtypeStruct((4, T_PAD, H), jnp.float32),
    )(x_pad, w, b)


def _sc_aggregate(wh_flat, edges, src_off, zrow, zcnt, ones):
    mesh = plsc.VectorSubcoreMesh(core_axis_name="c", subcore_axis_name="s")

    @functools.partial(
        pl.kernel,
        out_type=[
            jax.ShapeDtypeStruct((2, A_PAD, H), jnp.float32),
            jax.ShapeDtypeStruct((2, A_PAD), jnp.float32),
        ],
        mesh=mesh,
        compiler_params=pltpu.CompilerParams(use_tc_tiling_on_sc=False),
        scratch_types=[
            pltpu.VMEM((N_CHUNKS, CHUNK), jnp.int32),
            pltpu.VMEM((N_CHUNKS, CHUNK), jnp.int32),
            pltpu.VMEM((N_CHUNKS, CHUNK), jnp.int32),
            pltpu.VMEM((CHUNK, H), jnp.float32),
            pltpu.VMEM((SLAB, H), jnp.float32),
            pltpu.VMEM((DEG_RUN,), jnp.float32),
            pltpu.VMEM((CHUNK,), jnp.float32),
            pltpu.VMEM_SHARED((A_PAD, H), jnp.float32),
            pltpu.VMEM_SHARED((A_PAD,), jnp.float32),
            pltpu.SemaphoreType.DMA,
        ],
    )
    def agg(wh_hbm, edges_hbm, srcoff_hbm, zrow_hbm, zcnt_hbm, ones_hbm,
            g_out, deg_out,
            srcg_v, dst_v, srcr_v, rows_v, zbuf, cbuf, ones_v,
            acc_sh, deg_sh, sem):
        c = lax.axis_index("c")
        s = lax.axis_index("s")

        pltpu.sync_copy(zrow_hbm, zbuf)
        pltpu.sync_copy(zcnt_hbm, cbuf)
        pltpu.sync_copy(ones_hbm, ones_v)

        base = s * ROWS_PER_TILE
        for k in range(4):
            off = base + k * SLAB
            pltpu.sync_copy(zbuf, acc_sh.at[pl.ds(off, SLAB)])

        @pl.when(s == 0)
        def _():
            for k in range(4):
                pltpu.sync_copy(cbuf, deg_sh.at[pl.ds(k * DEG_RUN, DEG_RUN)])
        plsc.subcore_barrier()

        for t in range(2):
            pltpu.sync_copy(srcoff_hbm.at[c, t, s], srcg_v)
            pltpu.sync_copy(edges_hbm.at[t, 1, s], dst_v)

            def chunk_body(j, carry):
                pltpu.async_copy(wh_hbm.at[srcg_v.at[j]], rows_v, sem).wait()
                pltpu.sync_copy(rows_v, acc_sh.at[dst_v.at[j]], add=True)
                return carry

            lax.fori_loop(0, N_CHUNKS, chunk_body, 0)

            @pl.when(c == t)
            def _():
                pltpu.sync_copy(edges_hbm.at[t, 0, s], srcr_v)

                def deg_body(j, carry):
                    pltpu.sync_copy(ones_v, deg_sh.at[srcr_v.at[j]],
                                    add=True)
                    return carry

                lax.fori_loop(0, N_CHUNKS, deg_body, 0)

        plsc.subcore_barrier()

        for k in range(4):
            off = base + k * SLAB
            pltpu.sync_copy(acc_sh.at[pl.ds(off, SLAB)],
                            g_out.at[c, pl.ds(off, SLAB)])

        @pl.when(s % 4 == 0)
        def _():
            run = pl.multiple_of((s // 4) * DEG_RUN, DEG_RUN)
            pltpu.sync_copy(deg_sh.at[pl.ds(run, DEG_RUN)],
                            deg_out.at[c, pl.ds(run, DEG_RUN)])

    return agg(wh_flat, edges, src_off, zrow, zcnt, ones)


def _tc_finalize(g, deg):
    blk = 1000

    def body(g_ref, deg_ref, o_ref):
        d = deg_ref[0] + deg_ref[1]
        inv = 1.0 / jnp.where(d == 0.0, 1.0, d)
        acc = jnp.concatenate([g_ref[0], g_ref[1]], axis=1) * inv
        o_ref[...] = jnp.maximum(acc, 0.0)

    return pl.pallas_call(
        body,
        grid=(N_NODES // blk,),
        in_specs=[
            pl.BlockSpec((2, blk, H), lambda i: (0, i, 0)),
            pl.BlockSpec((2, blk, 1), lambda i: (0, i, 0)),
        ],
        out_specs=pl.BlockSpec((blk, D), lambda i: (i, 0)),
        out_shape=jax.ShapeDtypeStruct((N_NODES, D), jnp.float32),
    )(g, deg)


def _prep_edges(ei):
    p = jnp.pad(ei, ((0, 0), (0, E_PAD - N_EDGES)), constant_values=PAD_IDX)
    return p.reshape(2, 16, N_CHUNKS, CHUNK)


@jax.jit
def kernel(x, edge_index_fwd, edge_index_bwd, W_af, b_af, W_ab, b_ab):
    x_pad = jnp.pad(x, ((0, T_PAD - N_NODES), (0, 0)))
    edges = jnp.stack([_prep_edges(edge_index_fwd),
                       _prep_edges(edge_index_bwd)])
    src = edges[:, 0]
    src_off = jnp.stack([
        jnp.stack([src[0] + 0 * T_PAD, src[1] + 2 * T_PAD]),
        jnp.stack([src[0] + 1 * T_PAD, src[1] + 3 * T_PAD]),
    ])
    zrow = jnp.zeros((SLAB, H), jnp.float32)
    zcnt = jnp.zeros((DEG_RUN,), jnp.float32)
    ones = jnp.ones((CHUNK,), jnp.float32)

    w = jnp.stack([W_af, W_ab])
    b = jnp.stack([b_af, b_ab])
    wh = _tc_project(x_pad, w, b).reshape(4 * T_PAD, H)

    g, deg = _sc_aggregate(wh, edges, src_off, zrow, zcnt, ones)
    return _tc_finalize(g, deg.reshape(2, A_PAD, 1))
